# baseline (speedup 1.0000x reference)
import functools

import jax
import jax.numpy as jnp
from jax import lax
from jax.experimental import pallas as pl
from jax.experimental.pallas import tpu as pltpu
from jax.experimental.pallas import tpu_sc as plsc

VOCAB = 1000000
DIM = 64
B = 4096
L = 200

NC = 2
NS = 16
NW = NC * NS
BPW = B // NW
NCHUNK = 2
CH = L // NCHUNK
CHP = 128
LANES = 16
NBUF = 2
UNROLL = 4
RB = 8192


def _build_convert_kernel():
  def body(wt_ref, out_ref):
    blk = wt_ref[...].T
    out_ref[...] = jnp.concatenate([blk, blk], axis=1)

  return pl.pallas_call(
      body,
      grid=(pl.cdiv(VOCAB, RB),),
      in_specs=[pl.BlockSpec((DIM, RB), lambda i: (0, i))],
      out_specs=pl.BlockSpec((RB, 2 * DIM), lambda i: (i, 0)),
      out_shape=jax.ShapeDtypeStruct((VOCAB, 2 * DIM), jnp.float32),
  )


def _build_gather_kernel():
  mesh = plsc.VectorSubcoreMesh(core_axis_name="c", subcore_axis_name="s")

  @functools.partial(
      pl.kernel,
      mesh=mesh,
      compiler_params=pltpu.CompilerParams(
          use_tc_tiling_on_sc=True, needs_layout_passes=False),
      out_type=jax.ShapeDtypeStruct((B, 2 * DIM), jnp.float32),
      scratch_types=[
          pltpu.VMEM((NCHUNK * BPW, CHP), jnp.int32),
          pltpu.VMEM((BPW,), jnp.float32),
          pltpu.VMEM((NBUF, L, 2 * DIM), jnp.float32),
          pltpu.VMEM((BPW, 2 * DIM), jnp.float32),
          pltpu.SemaphoreType.DMA,
          pltpu.SemaphoreType.DMA,
      ],
  )
  def k2(q_hbm, len_hbm, wd_hbm, out_hbm,
         idx_v, len_v, rows_v, out_v, *sems):
    wid = lax.axis_index("s") * NC + lax.axis_index("c")
    base = wid * BPW

    pltpu.sync_copy(q_hbm.at[pl.ds(NCHUNK * base, NCHUNK * BPW)], idx_v)
    pltpu.sync_copy(len_hbm.at[pl.ds(base, BPW)], len_v)

    def start_gather(b, j):
      pltpu.async_copy(
          wd_hbm.at[idx_v.at[NCHUNK * b, pl.ds(0, CH)]],
          rows_v.at[j, pl.ds(0, CH)], sems[j])
      pltpu.async_copy(
          wd_hbm.at[idx_v.at[NCHUNK * b + 1, pl.ds(0, CH)]],
          rows_v.at[j, pl.ds(CH, CH)], sems[j])

    def wait_gather(j):
      pltpu.make_async_copy(
          wd_hbm.at[pl.ds(0, L)], rows_v.at[j], sems[j]).wait()

    for j in range(NBUF):
      start_gather(j, j)

    def group_body(g, _):
      for j in range(NBUF):
        b = g * NBUF + j
        wait_gather(j)

        zero = jnp.zeros((LANES,), jnp.float32)

        def red(i, accs):
          a0, a1, a2, a3 = accs
          l0 = i * UNROLL
          for r in range(UNROLL):
            a0 = a0 + rows_v[j, l0 + r, pl.ds(0 * LANES, LANES)]
            a1 = a1 + rows_v[j, l0 + r, pl.ds(1 * LANES, LANES)]
            a2 = a2 + rows_v[j, l0 + r, pl.ds(2 * LANES, LANES)]
            a3 = a3 + rows_v[j, l0 + r, pl.ds(3 * LANES, LANES)]
          return (a0, a1, a2, a3)

        a0, a1, a2, a3 = lax.fori_loop(
            0, L // UNROLL, red, (zero, zero, zero, zero))

        start_gather(jnp.minimum(b + NBUF, BPW - 1), j)

        grp = (b // LANES) * LANES
        lv = len_v[pl.ds(grp, LANES)]
        lenb = lax.gather(
            lv, jnp.full((LANES, 1), b - grp, jnp.int32),
            lax.GatherDimensionNumbers(
                offset_dims=(), collapsed_slice_dims=(0,),
                start_index_map=(0,)),
            (1,), mode=lax.GatherScatterMode.PROMISE_IN_BOUNDS)
        inv = 1.0 / lenb
        out_v[b, pl.ds(0 * LANES, LANES)] = a0 * inv
        out_v[b, pl.ds(1 * LANES, LANES)] = a1 * inv
        out_v[b, pl.ds(2 * LANES, LANES)] = a2 * inv
        out_v[b, pl.ds(3 * LANES, LANES)] = a3 * inv
      return 0

    lax.fori_loop(0, BPW // NBUF, group_body, 0)

    for j in range(NBUF):
      wait_gather(j)

    pltpu.sync_copy(out_v, out_hbm.at[pl.ds(base, BPW)])

  return k2


_convert = _build_convert_kernel()
_gather = _build_gather_kernel()


def kernel(query, query_length, W):
  q = query.reshape(NCHUNK * B, CH)
  q = jnp.pad(q, ((0, 0), (0, CHP - CH)))
  lens = query_length.astype(jnp.float32)
  wd = _convert(W.T)
  out = _gather(q, lens, wd)
  return out[:, :DIM]


# speedup vs baseline: 1.0639x; 1.0639x over previous
import functools

import jax
import jax.numpy as jnp
from jax import lax
from jax.experimental import pallas as pl
from jax.experimental.pallas import tpu as pltpu
from jax.experimental.pallas import tpu_sc as plsc

VOCAB = 1000000
DIM = 64
B = 4096
L = 200

NC = 2
NS = 16
NW = NC * NS
BPW = B // NW
NCHUNK = 2
CH = L // NCHUNK
CHP = 128
LANES = 16
NBUF = 2
UNROLL = 4
RB = 16384


def _build_convert_kernel():
  def body(wt_ref, out_ref):
    blk = wt_ref[...].T
    out_ref[...] = jnp.concatenate([blk, blk], axis=1)

  return pl.pallas_call(
      body,
      grid=(pl.cdiv(VOCAB, RB),),
      in_specs=[pl.BlockSpec((DIM, RB), lambda i: (0, i))],
      out_specs=pl.BlockSpec((RB, 2 * DIM), lambda i: (i, 0)),
      out_shape=jax.ShapeDtypeStruct((VOCAB, 2 * DIM), jnp.float32),
  )


def _build_gather_kernel():
  mesh = plsc.VectorSubcoreMesh(core_axis_name="c", subcore_axis_name="s")

  @functools.partial(
      pl.kernel,
      mesh=mesh,
      compiler_params=pltpu.CompilerParams(
          use_tc_tiling_on_sc=True, needs_layout_passes=False),
      out_type=jax.ShapeDtypeStruct((B, 2 * DIM), jnp.float32),
      scratch_types=[
          pltpu.VMEM((NCHUNK * BPW, CHP), jnp.int32),
          pltpu.VMEM((BPW,), jnp.float32),
          pltpu.VMEM((NBUF, L, 2 * DIM), jnp.float32),
          pltpu.VMEM((BPW, 2 * DIM), jnp.float32),
          pltpu.SemaphoreType.DMA,
          pltpu.SemaphoreType.DMA,
          pltpu.SemaphoreType.DMA,
      ],
  )
  def k2(q_hbm, len_hbm, wd_hbm, out_hbm,
         idx_v, len_v, rows_v, out_v, *sems):
    wid = lax.axis_index("s") * NC + lax.axis_index("c")
    base = wid * BPW

    pltpu.sync_copy(q_hbm.at[pl.ds(NCHUNK * base, NCHUNK * BPW)], idx_v)
    pltpu.sync_copy(len_hbm.at[pl.ds(base, BPW)], len_v)

    def start_gather(b, j):
      pltpu.async_copy(
          wd_hbm.at[idx_v.at[NCHUNK * b, pl.ds(0, CH)]],
          rows_v.at[j, pl.ds(0, CH)], sems[j])
      pltpu.async_copy(
          wd_hbm.at[idx_v.at[NCHUNK * b + 1, pl.ds(0, CH)]],
          rows_v.at[j, pl.ds(CH, CH)], sems[j])

    def wait_gather(j):
      pltpu.make_async_copy(
          wd_hbm.at[pl.ds(0, L)], rows_v.at[j], sems[j]).wait()

    for j in range(NBUF):
      start_gather(j, j)

    def group_body(g, _):
      for j in range(NBUF):
        b = g * NBUF + j
        wait_gather(j)

        zero = jnp.zeros((LANES,), jnp.float32)

        def red(i, accs):
          a0, a1, a2, a3 = accs
          l0 = i * UNROLL
          for r in range(UNROLL):
            a0 = a0 + rows_v[j, l0 + r, pl.ds(0 * LANES, LANES)]
            a1 = a1 + rows_v[j, l0 + r, pl.ds(1 * LANES, LANES)]
            a2 = a2 + rows_v[j, l0 + r, pl.ds(2 * LANES, LANES)]
            a3 = a3 + rows_v[j, l0 + r, pl.ds(3 * LANES, LANES)]
          return (a0, a1, a2, a3)

        a0, a1, a2, a3 = lax.fori_loop(
            0, L // UNROLL, red, (zero, zero, zero, zero))

        start_gather(jnp.minimum(b + NBUF, BPW - 1), j)

        grp = (b // LANES) * LANES
        lv = len_v[pl.ds(grp, LANES)]
        lenb = lax.gather(
            lv, jnp.full((LANES, 1), b - grp, jnp.int32),
            lax.GatherDimensionNumbers(
                offset_dims=(), collapsed_slice_dims=(0,),
                start_index_map=(0,)),
            (1,), mode=lax.GatherScatterMode.PROMISE_IN_BOUNDS)
        inv = 1.0 / lenb
        out_v[b, pl.ds(0 * LANES, LANES)] = a0 * inv
        out_v[b, pl.ds(1 * LANES, LANES)] = a1 * inv
        out_v[b, pl.ds(2 * LANES, LANES)] = a2 * inv
        out_v[b, pl.ds(3 * LANES, LANES)] = a3 * inv
      return 0

    lax.fori_loop(0, BPW // NBUF, group_body, 0)

    for j in range(NBUF):
      wait_gather(j)

    pltpu.sync_copy(out_v, out_hbm.at[pl.ds(base, BPW)])

  return k2


_convert = _build_convert_kernel()
_gather = _build_gather_kernel()


def kernel(query, query_length, W):
  q = query.reshape(NCHUNK * B, CH)
  q = jnp.pad(q, ((0, 0), (0, CHP - CH)))
  lens = query_length.astype(jnp.float32)
  wd = _convert(W.T)
  out = _gather(q, lens, wd)
  return out[:, :DIM]
